# Initial kernel scaffold; baseline (speedup 1.0000x reference)
import jax
import jax.numpy as jnp
from jax.experimental import pallas as pl


def kernel(x, a_row_ids, a_cols, a_vals, b_row_ids, b_cols, b_vals):
    raise NotImplementedError("write your pallas kernel here")


# speedup vs baseline: 14.2197x; 14.2197x over previous
import functools

import jax
import jax.numpy as jnp
from jax import lax
from jax.experimental import pallas as pl
from jax.experimental.pallas import tpu as pltpu
from jax.experimental.pallas import tpu_sc as plsc

NNZ = 41
BATCH = 64
NC = 2
NS = 16
NW = NC * NS
RPG = 2
GIDX = RPG * NNZ
LANES = 16
CHUNKS = BATCH // LANES
NNZ_PAD = 48
VROW = RPG * NNZ_PAD


def _stage(nrows):
    groups = nrows // RPG
    gpw = groups // NW
    rpw = nrows // NW

    mesh = plsc.VectorSubcoreMesh(core_axis_name="c", subcore_axis_name="s")

    @functools.partial(
        pl.kernel,
        mesh=mesh,
        out_type=jax.ShapeDtypeStruct((nrows, BATCH), jnp.float32),
        compiler_params=pltpu.CompilerParams(use_tc_tiling_on_sc=False),
        scratch_types=[
            pltpu.VMEM((gpw, GIDX), jnp.int32),
            pltpu.VMEM((gpw, VROW), jnp.float32),
            pltpu.VMEM((GIDX, BATCH), jnp.float32),
            pltpu.VMEM((GIDX, BATCH), jnp.float32),
            pltpu.VMEM((rpw, BATCH), jnp.float32),
            pltpu.SemaphoreType.DMA,
            pltpu.SemaphoreType.DMA,
        ],
    )
    def stage(table, cols, vals, out, cols_v, vals_v, buf0, buf1, out_v,
              sem0, sem1):
        wid = lax.axis_index("s") * NC + lax.axis_index("c")
        gbase = wid * gpw

        pltpu.sync_copy(cols.at[pl.ds(gbase, gpw)], cols_v)
        pltpu.sync_copy(vals.at[pl.ds(gbase, gpw)], vals_v)

        bufs = (buf0, buf1)
        sems = (sem0, sem1)

        pltpu.make_async_copy(table.at[cols_v.at[0]], buf0, sem0).start()

        def body(i, carry):
            for b in range(2):
                g = 2 * i + b
                buf, sem = bufs[b], sems[b]
                pltpu.make_async_copy(table.at[cols_v.at[g]], buf, sem).wait()

                nxt = g + 1

                @pl.when(nxt < gpw)
                def _():
                    pltpu.make_async_copy(
                        table.at[cols_v.at[nxt]], bufs[1 - b], sems[1 - b]
                    ).start()

                for r in range(RPG):
                    acc = [jnp.zeros((LANES,), jnp.float32)
                           for _ in range(CHUNKS)]
                    vv = [vals_v[g, pl.ds(r * NNZ_PAD + k * LANES, LANES)]
                          for k in range(NNZ_PAD // LANES)]
                    for j in range(NNZ):
                        e = r * NNZ + j
                        v = vv[j // LANES][j % LANES]
                        for c in range(CHUNKS):
                            acc[c] = acc[c] + v * buf[e, pl.ds(c * LANES,
                                                               LANES)]
                    for c in range(CHUNKS):
                        out_v[RPG * g + r, pl.ds(c * LANES, LANES)] = acc[c]
            return carry

        lax.fori_loop(0, gpw // 2, body, 0)
        pltpu.sync_copy(out_v, out.at[pl.ds(wid * rpw, rpw)])

    return stage


def _csr_matvec(table, cols, vals):
    nrows = vals.shape[0] // NNZ
    cols2 = cols.reshape(nrows // RPG, GIDX)
    vals2 = jnp.pad(vals.reshape(nrows, NNZ), ((0, 0), (0, NNZ_PAD - NNZ)))
    vals2 = vals2.reshape(nrows // RPG, VROW)
    return _stage(nrows)(table, cols2, vals2)


def kernel(x, a_row_ids, a_cols, a_vals, b_row_ids, b_cols, b_vals):
    t = _csr_matvec(x, b_cols, b_vals)
    y = _csr_matvec(t, a_cols, a_vals)
    return jnp.transpose(y)[None, :, :]


# speedup vs baseline: 25.6118x; 1.8011x over previous
import functools

import jax
import jax.numpy as jnp
from jax import lax
from jax.experimental import pallas as pl
from jax.experimental.pallas import tpu as pltpu
from jax.experimental.pallas import tpu_sc as plsc

NNZ = 41
BATCH = 64
NC = 2
NS = 16
NW = NC * NS
RPG = 2
GIDX = RPG * NNZ
LANES = 16
CHUNKS = BATCH // LANES
VCHUNK_OFF = (0, 16, 32, 48, 64, 66)


def _wchunk(j):
    if j < 80:
        return j // 16, j % 16
    return 5, j - 66


def _stage(nrows, tab_rows):
    groups = nrows // RPG
    gpw = groups // NW
    rpw = nrows // NW
    tab_per_sub = tab_rows // NS

    mesh = plsc.VectorSubcoreMesh(core_axis_name="c", subcore_axis_name="s")

    @functools.partial(
        pl.kernel,
        mesh=mesh,
        out_type=jax.ShapeDtypeStruct((nrows, BATCH), jnp.float32),
        compiler_params=pltpu.CompilerParams(use_tc_tiling_on_sc=False),
        scratch_types=[
            pltpu.VMEM_SHARED((tab_rows, BATCH), jnp.float32),
            pltpu.VMEM((gpw, GIDX), jnp.int32),
            pltpu.VMEM((gpw, GIDX), jnp.float32),
            pltpu.VMEM((GIDX, BATCH), jnp.float32),
            pltpu.VMEM((GIDX, BATCH), jnp.float32),
            pltpu.VMEM((rpw, BATCH), jnp.float32),
            pltpu.SemaphoreType.DMA,
            pltpu.SemaphoreType.DMA,
        ],
    )
    def stage(table, cols, vals, out, tab_s, cols_v, vals_v, buf0, buf1,
              out_v, sem0, sem1):
        sid = lax.axis_index("s")
        wid = sid * NC + lax.axis_index("c")
        gbase = wid * gpw

        pltpu.sync_copy(table.at[pl.ds(sid * tab_per_sub, tab_per_sub)],
                        tab_s.at[pl.ds(sid * tab_per_sub, tab_per_sub)])
        pltpu.sync_copy(cols.at[pl.ds(gbase, gpw)], cols_v)
        pltpu.sync_copy(vals.at[pl.ds(gbase, gpw)], vals_v)
        plsc.subcore_barrier()

        bufs = (buf0, buf1)
        sems = (sem0, sem1)

        pltpu.make_async_copy(tab_s.at[cols_v.at[0]], buf0, sem0).start()

        def body(i, carry):
            for b in range(2):
                g = 2 * i + b
                buf, sem = bufs[b], sems[b]
                pltpu.make_async_copy(tab_s.at[cols_v.at[g]], buf, sem).wait()

                nxt = g + 1

                @pl.when(nxt < gpw)
                def _():
                    pltpu.make_async_copy(
                        tab_s.at[cols_v.at[nxt]], bufs[1 - b], sems[1 - b]
                    ).start()

                for r in range(RPG):
                    acc = [jnp.zeros((LANES,), jnp.float32)
                           for _ in range(CHUNKS)]
                    vv = [vals_v[g, pl.ds(off, LANES)] for off in VCHUNK_OFF]
                    for j in range(NNZ):
                        e = r * NNZ + j
                        ck, lane = _wchunk(e)
                        v = vv[ck][lane]
                        for c in range(CHUNKS):
                            acc[c] = acc[c] + v * buf[e, pl.ds(c * LANES,
                                                               LANES)]
                    for c in range(CHUNKS):
                        out_v[RPG * g + r, pl.ds(c * LANES, LANES)] = acc[c]
            return carry

        lax.fori_loop(0, gpw // 2, body, 0)
        pltpu.sync_copy(out_v, out.at[pl.ds(wid * rpw, rpw)])

    return stage


def _csr_matvec(table, cols, vals):
    nrows = vals.shape[0] // NNZ
    cols2 = cols.reshape(nrows // RPG, GIDX)
    vals2 = vals.reshape(nrows // RPG, GIDX)
    return _stage(nrows, table.shape[0])(table, cols2, vals2)


def kernel(x, a_row_ids, a_cols, a_vals, b_row_ids, b_cols, b_vals):
    t = _csr_matvec(x, b_cols, b_vals)
    y = _csr_matvec(t, a_cols, a_vals)
    return jnp.transpose(y)[None, :, :]
